# baseline (speedup 1.0000x reference)
import functools

import jax
import jax.numpy as jnp
from jax import lax
from jax.experimental import pallas as pl
from jax.experimental.pallas import tpu as pltpu
from jax.experimental.pallas import tpu_sc as plsc

ROWB = 1000
CHW = 16
GRAN = 128
BLKG = 14
NPAD = 50048


def _cdiv(a, b):
    return (a + b - 1) // b


def _k_input(x_ref, w_ref, b_ref, h_ref, s1_ref, s2_ref, mn_ref, mx_ref):
    h = jnp.dot(x_ref[...], w_ref[...], preferred_element_type=jnp.float32)
    h = h + b_ref[...]
    h_ref[...] = h

    @pl.when(pl.program_id(0) == 0)
    def _():
        s1_ref[...] = jnp.zeros_like(s1_ref)
        s2_ref[...] = jnp.zeros_like(s2_ref)
        mn_ref[...] = jnp.full_like(mn_ref, jnp.inf)
        mx_ref[...] = jnp.full_like(mx_ref, -jnp.inf)

    s1_ref[...] += jnp.sum(h, axis=0, keepdims=True)
    s2_ref[...] += jnp.sum(h * h, axis=0, keepdims=True)
    mn_ref[...] = jnp.minimum(mn_ref[...], jnp.min(h, axis=0, keepdims=True))
    mx_ref[...] = jnp.maximum(mx_ref[...], jnp.max(h, axis=0, keepdims=True))


def _k_preptab(act_i, h_ref, sc_ref, sh_ref, m_ref, hn_ref, tab_ref):
    v = h_ref[...] * sc_ref[...] + sh_ref[...]
    if act_i == 0:
        hn = jnp.where(v >= 0, v, 0.01 * v)
    else:
        hn = jnp.tanh(v)
    hn_ref[...] = hn
    r = jnp.maximum(hn, 0.0) + 1e-7
    t = jnp.exp(r - m_ref[...])
    u = r * t
    tab_ref[...] = jnp.concatenate([t, u], axis=1)


def _k_mlp1(ch, acc_ref, hn_ref, w_ref, b_ref, z_ref, s1_ref, s2_ref):
    a = acc_ref[...]
    den = a[:, 0:ch]
    num = a[:, ch:2 * ch]
    safe = jnp.where(den > 0, den, 1.0)
    agg = jnp.where(den > 0, num / safe, 0.0)
    out = agg + hn_ref[...]
    z = jnp.dot(out, w_ref[...], preferred_element_type=jnp.float32)
    z = z + b_ref[...]
    z_ref[...] = z

    @pl.when(pl.program_id(0) == 0)
    def _():
        s1_ref[...] = jnp.zeros_like(s1_ref)
        s2_ref[...] = jnp.zeros_like(s2_ref)

    s1_ref[...] += jnp.sum(z, axis=0, keepdims=True)
    s2_ref[...] += jnp.sum(z * z, axis=0, keepdims=True)


def _k_mlp2(z1_ref, sc_ref, sh_ref, w_ref, b_ref, z_ref, s1_ref, s2_ref):
    a = jnp.maximum(z1_ref[...] * sc_ref[...] + sh_ref[...], 0.0)
    z = jnp.dot(a, w_ref[...], preferred_element_type=jnp.float32)
    z = z + b_ref[...]
    z_ref[...] = z

    @pl.when(pl.program_id(0) == 0)
    def _():
        s1_ref[...] = jnp.zeros_like(s1_ref)
        s2_ref[...] = jnp.zeros_like(s2_ref)

    s1_ref[...] += jnp.sum(z, axis=0, keepdims=True)
    s2_ref[...] += jnp.sum(z * z, axis=0, keepdims=True)


def _k_mlp3enc(z2_ref, sc_ref, sh_ref, w3_ref, b3_ref, h_ref, we_ref, be_ref,
               ho_ref, s1_ref, s2_ref, mn_ref, mx_ref):
    a = jnp.maximum(z2_ref[...] * sc_ref[...] + sh_ref[...], 0.0)
    z3 = jnp.dot(a, w3_ref[...], preferred_element_type=jnp.float32)
    hm = h_ref[...] + z3 + b3_ref[...]
    ho = jnp.dot(hm, we_ref[...], preferred_element_type=jnp.float32)
    ho = ho + be_ref[...]
    ho_ref[...] = ho

    @pl.when(pl.program_id(0) == 0)
    def _():
        s1_ref[...] = jnp.zeros_like(s1_ref)
        s2_ref[...] = jnp.zeros_like(s2_ref)
        mn_ref[...] = jnp.full_like(mn_ref, jnp.inf)
        mx_ref[...] = jnp.full_like(mx_ref, -jnp.inf)

    s1_ref[...] += jnp.sum(ho, axis=0, keepdims=True)
    s2_ref[...] += jnp.sum(ho * ho, axis=0, keepdims=True)
    mn_ref[...] = jnp.minimum(mn_ref[...], jnp.min(ho, axis=0, keepdims=True))
    mx_ref[...] = jnp.maximum(mx_ref[...], jnp.max(ho, axis=0, keepdims=True))


def _k_pool(ngroups, bat_ref, h_ref, ps_ref, cnt_ref):
    @pl.when(pl.program_id(0) == 0)
    def _():
        ps_ref[...] = jnp.zeros_like(ps_ref)
        cnt_ref[...] = jnp.zeros_like(cnt_ref)

    gi = lax.broadcasted_iota(jnp.int32, (ngroups, ROWB), 0)
    oh = (gi == bat_ref[0]).astype(jnp.float32)
    ps_ref[...] += jnp.dot(oh, h_ref[...], preferred_element_type=jnp.float32)
    cnt_ref[...] += jnp.sum(oh, axis=1, keepdims=True)


def _k_final(ps_ref, cnt_ref, w_ref, b_ref, o_ref):
    pooled = ps_ref[...] / jnp.maximum(cnt_ref[...], 1.0)
    o = jnp.dot(pooled, w_ref[...], preferred_element_type=jnp.float32)
    o_ref[...] = o + b_ref[...]


def _rows(n):
    return pl.BlockSpec((ROWB, n), lambda k: (k, 0))


def _bcast(r, c):
    return pl.BlockSpec((r, c), lambda k: (0, 0))


def _bn_affine(s1, s2, g, b, n):
    mean = s1 / n
    var = s2 / n - mean * mean
    rs = lax.rsqrt(var + 1e-5)
    scale = rs * g[None, :]
    shift = b[None, :] - mean * scale
    return scale, shift


@functools.cache
def _make_edge_kernel(nchunks, rows_total):
    npsc = nchunks // 2
    rows_tile = rows_total // 16
    nblk = rows_tile // BLKG
    half = nblk // 2
    eb = BLKG * GRAN
    rpt = NPAD // 16
    zrows = rpt // 8
    mesh = plsc.VectorSubcoreMesh(core_axis_name="c", subcore_axis_name="s")

    @functools.partial(
        pl.kernel,
        out_type=jax.ShapeDtypeStruct((NPAD, nchunks * CHW), jnp.float32),
        mesh=mesh,
        scratch_types=[
            pltpu.VMEM_SHARED((NPAD, CHW), jnp.float32),
            pltpu.VMEM((BLKG, 2, GRAN), jnp.int32),
            pltpu.VMEM((BLKG, 2, GRAN), jnp.int32),
            pltpu.VMEM((eb, CHW), jnp.float32),
            pltpu.VMEM((eb, CHW), jnp.float32),
            pltpu.VMEM((zrows, CHW), jnp.float32),
            pltpu.SemaphoreType.DMA,
            pltpu.SemaphoreType.DMA,
            pltpu.SemaphoreType.DMA,
            pltpu.SemaphoreType.DMA,
            pltpu.SemaphoreType.DMA,
            pltpu.SemaphoreType.DMA,
        ],
        compiler_params=pltpu.CompilerParams(use_tc_tiling_on_sc=False),
    )
    def kern(table, edg, out, acc, idx_a, idx_b, gbuf_a, gbuf_b, zbuf,
             sem_ga, sem_gb, sem_sa, sem_sb, sem_ia, sem_ib):
        c = lax.axis_index("c")
        s = lax.axis_index("s")
        rbase = s * rpt
        ebase = s * rows_tile
        ncv = jnp.broadcast_to(jnp.int32(nchunks), (16,))
        zv = jnp.zeros((16,), jnp.float32)

        def zfill(i, carry):
            zbuf[i, pl.ds(0, CHW)] = zv
            return carry

        lax.fori_loop(0, zrows, zfill, 0)

        def fire_idx(bidx, idx, sem):
            row = ebase + bidx * BLKG
            pltpu.async_copy(edg.at[pl.ds(row, BLKG)], idx, sem)

        def finish_idx(bidx, idx, sem, chv):
            row = ebase + bidx * BLKG
            pltpu.make_async_copy(edg.at[pl.ds(row, BLKG)], idx, sem).wait()
            for j in range(BLKG):
                for l in range(GRAN // 16):
                    sl = pl.ds(l * 16, 16)
                    idx[j, 0, sl] = idx[j, 0, sl] + chv

        def fire_g(idx, gbuf, sem):
            for j in range(BLKG):
                pltpu.async_copy(table.at[idx.at[j, 0]],
                                 gbuf.at[pl.ds(j * GRAN, GRAN)], sem)

        def wait_g(idx, gbuf, sem):
            for j in range(BLKG):
                pltpu.make_async_copy(table.at[idx.at[j, 0]],
                                      gbuf.at[pl.ds(j * GRAN, GRAN)],
                                      sem).wait()

        def fire_s(idx, gbuf, sem):
            for j in range(BLKG):
                pltpu.async_copy(gbuf.at[pl.ds(j * GRAN, GRAN)],
                                 acc.at[idx.at[j, 1]], sem, add=True)

        def wait_s(idx, gbuf, sem):
            for j in range(BLKG):
                pltpu.make_async_copy(gbuf.at[pl.ds(j * GRAN, GRAN)],
                                      acc.at[idx.at[j, 1]], sem).wait()

        for ci in range(npsc):
            chunk = c * npsc + ci
            chv = jnp.broadcast_to(chunk, (16,)).astype(jnp.int32)
            for j in range(8):
                pltpu.sync_copy(zbuf, acc.at[pl.ds(rbase + j * zrows, zrows)])
            plsc.subcore_barrier()

            fire_idx(0, idx_a, sem_ia)
            finish_idx(0, idx_a, sem_ia, chv)
            fire_g(idx_a, gbuf_a, sem_ga)
            fire_idx(1, idx_b, sem_ib)

            def pair(i, carry):
                wait_g(idx_a, gbuf_a, sem_ga)
                fire_s(idx_a, gbuf_a, sem_sa)

                @pl.when(i > 0)
                def _():
                    wait_s(idx_b, gbuf_b, sem_sb)
                    fire_idx(2 * i + 1, idx_b, sem_ib)

                wait_s(idx_a, gbuf_a, sem_sa)
                finish_idx(2 * i + 1, idx_b, sem_ib, chv)
                fire_g(idx_b, gbuf_b, sem_gb)

                @pl.when(i < half - 1)
                def _():
                    fire_idx(2 * i + 2, idx_a, sem_ia)
                    finish_idx(2 * i + 2, idx_a, sem_ia, chv)
                    fire_g(idx_a, gbuf_a, sem_ga)

                wait_g(idx_b, gbuf_b, sem_gb)
                fire_s(idx_b, gbuf_b, sem_sb)
                return carry

            lax.fori_loop(0, half, pair, 0)
            wait_s(idx_b, gbuf_b, sem_sb)
            plsc.subcore_barrier()
            pltpu.sync_copy(
                acc.at[pl.ds(rbase, rpt)],
                out.at[pl.ds(rbase, rpt), pl.ds(chunk * CHW, CHW)])

    return kern


def kernel(x, params, edge_index, batch):
    p = params
    n = x.shape[0]
    ngroups = 128
    nb = _cdiv(n, ROWB)

    f32 = jnp.float32
    sd = jax.ShapeDtypeStruct

    src = edge_index[0]
    dst = edge_index[1]
    e = src.shape[0]
    epb = 16 * 2 * BLKG * GRAN
    epad = _cdiv(e, epb) * epb
    srcp = jnp.concatenate([src, jnp.zeros((epad - e,), jnp.int32)])
    dstp = jnp.concatenate([dst, jnp.full((epad - e,), n, jnp.int32)])
    rows_total = epad // GRAN
    dst2d = dstp.reshape(rows_total, GRAN)

    def _edg(nchunks):
        return jnp.stack([(srcp * nchunks).reshape(rows_total, GRAN), dst2d],
                         axis=1)

    hdim = p["W_in"].shape[1]
    h, s1, s2, hmn, hmx = pl.pallas_call(
        _k_input,
        grid=(nb,),
        in_specs=[_rows(6), _bcast(6, hdim), _bcast(1, hdim)],
        out_specs=[_rows(hdim), _bcast(1, hdim), _bcast(1, hdim),
                   _bcast(1, hdim), _bcast(1, hdim)],
        out_shape=[sd((n, hdim), f32), sd((1, hdim), f32), sd((1, hdim), f32),
                   sd((1, hdim), f32), sd((1, hdim), f32)],
    )(x, p["W_in"], p["b_in"][None])

    for i, lp in enumerate(p["layers"]):
        ch = h.shape[1]
        nchunks = 2 * ch // CHW
        ch2 = 2 * ch

        scale, shift = _bn_affine(s1, s2, lp["n_g"], lp["n_b"], n)
        vext = scale * jnp.where(scale > 0, hmx, hmn) + shift
        an = jnp.where(vext >= 0, vext, 0.01 * vext) if i == 0 \
            else jnp.tanh(vext)
        m = jnp.maximum(jnp.max(an), 0.0) + 1e-7

        hn, tab = pl.pallas_call(
            functools.partial(_k_preptab, i),
            grid=(nb,),
            in_specs=[_rows(ch), _bcast(1, ch), _bcast(1, ch), _bcast(1, ch)],
            out_specs=[_rows(ch), _rows(ch2)],
            out_shape=[sd((n, ch), f32), sd((n, ch2), f32)],
        )(h, scale, shift, jnp.broadcast_to(m, (1, ch)))

        ek = _make_edge_kernel(nchunks, rows_total)
        acc2 = ek(tab.reshape(n * nchunks, CHW), _edg(nchunks))

        z1, s1a, s2a = pl.pallas_call(
            functools.partial(_k_mlp1, ch),
            grid=(nb,),
            in_specs=[_rows(ch2), _rows(ch), _bcast(ch, ch2), _bcast(1, ch2)],
            out_specs=[_rows(ch2), _bcast(1, ch2), _bcast(1, ch2)],
            out_shape=[sd((n, ch2), f32), sd((1, ch2), f32),
                       sd((1, ch2), f32)],
        )(acc2, hn, lp["m1W"], lp["m1b"][None])

        sc2, sh2 = _bn_affine(s1a, s2a, lp["m1g"], lp["m1be"], n)
        z2, s1b, s2b = pl.pallas_call(
            _k_mlp2,
            grid=(nb,),
            in_specs=[_rows(ch2), _bcast(1, ch2), _bcast(1, ch2),
                      _bcast(ch2, ch2), _bcast(1, ch2)],
            out_specs=[_rows(ch2), _bcast(1, ch2), _bcast(1, ch2)],
            out_shape=[sd((n, ch2), f32), sd((1, ch2), f32),
                       sd((1, ch2), f32)],
        )(z1, sc2, sh2, lp["m2W"], lp["m2b"][None])

        sc3, sh3 = _bn_affine(s1b, s2b, lp["m2g"], lp["m2be"], n)
        cho = ch // 2
        h, s1, s2, hmn, hmx = pl.pallas_call(
            _k_mlp3enc,
            grid=(nb,),
            in_specs=[_rows(ch2), _bcast(1, ch2), _bcast(1, ch2),
                      _bcast(ch2, ch), _bcast(1, ch), _rows(ch),
                      _bcast(ch, cho), _bcast(1, cho)],
            out_specs=[_rows(cho), _bcast(1, cho), _bcast(1, cho),
                       _bcast(1, cho), _bcast(1, cho)],
            out_shape=[sd((n, cho), f32), sd((1, cho), f32),
                       sd((1, cho), f32), sd((1, cho), f32),
                       sd((1, cho), f32)],
        )(z2, sc3, sh3, lp["m3W"], lp["m3b"][None], h, lp["eW"],
          lp["eb"][None])

    cho = h.shape[1]
    ps, cnt = pl.pallas_call(
        functools.partial(_k_pool, ngroups),
        grid=(nb,),
        in_specs=[pl.BlockSpec((1, 1, ROWB), lambda k: (k, 0, 0)), _rows(cho)],
        out_specs=[_bcast(ngroups, cho), _bcast(ngroups, 1)],
        out_shape=[sd((ngroups, cho), f32), sd((ngroups, 1), f32)],
    )(batch.astype(jnp.int32).reshape(nb, 1, ROWB), h)

    out = pl.pallas_call(
        _k_final,
        in_specs=[_bcast(ngroups, cho), _bcast(ngroups, 1),
                  _bcast(cho, 2), _bcast(1, 2)],
        out_specs=pl.BlockSpec((ngroups, 2), lambda k: (0, 0)),
        out_shape=sd((ngroups, 2), f32),
        grid=(1,),
    )(ps, cnt, p["W_lin"], p["b_lin"][None])
    return out


# speedup vs baseline: 1.0147x; 1.0147x over previous
import functools

import jax
import jax.numpy as jnp
from jax import lax
from jax.experimental import pallas as pl
from jax.experimental.pallas import tpu as pltpu
from jax.experimental.pallas import tpu_sc as plsc

ROWB = 1000
CHW = 16
GRAN = 128
BLKG = 14
NPAD = 50048


def _cdiv(a, b):
    return (a + b - 1) // b


def _k_input(x_ref, w_ref, b_ref, h_ref, s1_ref, s2_ref, mn_ref, mx_ref):
    h = jnp.dot(x_ref[...], w_ref[...], preferred_element_type=jnp.float32)
    h = h + b_ref[...]
    h_ref[...] = h

    @pl.when(pl.program_id(0) == 0)
    def _():
        s1_ref[...] = jnp.zeros_like(s1_ref)
        s2_ref[...] = jnp.zeros_like(s2_ref)
        mn_ref[...] = jnp.full_like(mn_ref, jnp.inf)
        mx_ref[...] = jnp.full_like(mx_ref, -jnp.inf)

    s1_ref[...] += jnp.sum(h, axis=0, keepdims=True)
    s2_ref[...] += jnp.sum(h * h, axis=0, keepdims=True)
    mn_ref[...] = jnp.minimum(mn_ref[...], jnp.min(h, axis=0, keepdims=True))
    mx_ref[...] = jnp.maximum(mx_ref[...], jnp.max(h, axis=0, keepdims=True))


def _k_preptab(act_i, h_ref, sc_ref, sh_ref, m_ref, hn_ref, tab_ref):
    v = h_ref[...] * sc_ref[...] + sh_ref[...]
    if act_i == 0:
        hn = jnp.where(v >= 0, v, 0.01 * v)
    else:
        hn = jnp.tanh(v)
    hn_ref[...] = hn
    r = jnp.maximum(hn, 0.0) + 1e-7
    t = jnp.exp(r - m_ref[...])
    u = r * t
    tab_ref[...] = jnp.concatenate([t, u], axis=1)


def _k_mlp1(ch, acc_ref, hn_ref, w_ref, b_ref, z_ref, s1_ref, s2_ref):
    a = acc_ref[...]
    den = a[:, 0:ch]
    num = a[:, ch:2 * ch]
    safe = jnp.where(den > 0, den, 1.0)
    agg = jnp.where(den > 0, num / safe, 0.0)
    out = agg + hn_ref[...]
    z = jnp.dot(out, w_ref[...], preferred_element_type=jnp.float32)
    z = z + b_ref[...]
    z_ref[...] = z

    @pl.when(pl.program_id(0) == 0)
    def _():
        s1_ref[...] = jnp.zeros_like(s1_ref)
        s2_ref[...] = jnp.zeros_like(s2_ref)

    s1_ref[...] += jnp.sum(z, axis=0, keepdims=True)
    s2_ref[...] += jnp.sum(z * z, axis=0, keepdims=True)


def _k_mlp2(z1_ref, sc_ref, sh_ref, w_ref, b_ref, z_ref, s1_ref, s2_ref):
    a = jnp.maximum(z1_ref[...] * sc_ref[...] + sh_ref[...], 0.0)
    z = jnp.dot(a, w_ref[...], preferred_element_type=jnp.float32)
    z = z + b_ref[...]
    z_ref[...] = z

    @pl.when(pl.program_id(0) == 0)
    def _():
        s1_ref[...] = jnp.zeros_like(s1_ref)
        s2_ref[...] = jnp.zeros_like(s2_ref)

    s1_ref[...] += jnp.sum(z, axis=0, keepdims=True)
    s2_ref[...] += jnp.sum(z * z, axis=0, keepdims=True)


def _k_mlp3enc(z2_ref, sc_ref, sh_ref, w3_ref, b3_ref, h_ref, we_ref, be_ref,
               ho_ref, s1_ref, s2_ref, mn_ref, mx_ref):
    a = jnp.maximum(z2_ref[...] * sc_ref[...] + sh_ref[...], 0.0)
    z3 = jnp.dot(a, w3_ref[...], preferred_element_type=jnp.float32)
    hm = h_ref[...] + z3 + b3_ref[...]
    ho = jnp.dot(hm, we_ref[...], preferred_element_type=jnp.float32)
    ho = ho + be_ref[...]
    ho_ref[...] = ho

    @pl.when(pl.program_id(0) == 0)
    def _():
        s1_ref[...] = jnp.zeros_like(s1_ref)
        s2_ref[...] = jnp.zeros_like(s2_ref)
        mn_ref[...] = jnp.full_like(mn_ref, jnp.inf)
        mx_ref[...] = jnp.full_like(mx_ref, -jnp.inf)

    s1_ref[...] += jnp.sum(ho, axis=0, keepdims=True)
    s2_ref[...] += jnp.sum(ho * ho, axis=0, keepdims=True)
    mn_ref[...] = jnp.minimum(mn_ref[...], jnp.min(ho, axis=0, keepdims=True))
    mx_ref[...] = jnp.maximum(mx_ref[...], jnp.max(ho, axis=0, keepdims=True))


def _k_pool(ngroups, bat_ref, h_ref, ps_ref, cnt_ref):
    @pl.when(pl.program_id(0) == 0)
    def _():
        ps_ref[...] = jnp.zeros_like(ps_ref)
        cnt_ref[...] = jnp.zeros_like(cnt_ref)

    gi = lax.broadcasted_iota(jnp.int32, (ngroups, ROWB), 0)
    oh = (gi == bat_ref[0]).astype(jnp.float32)
    ps_ref[...] += jnp.dot(oh, h_ref[...], preferred_element_type=jnp.float32)
    cnt_ref[...] += jnp.sum(oh, axis=1, keepdims=True)


def _k_final(ps_ref, cnt_ref, w_ref, b_ref, o_ref):
    pooled = ps_ref[...] / jnp.maximum(cnt_ref[...], 1.0)
    o = jnp.dot(pooled, w_ref[...], preferred_element_type=jnp.float32)
    o_ref[...] = o + b_ref[...]


def _rows(n):
    return pl.BlockSpec((ROWB, n), lambda k: (k, 0))


def _bcast(r, c):
    return pl.BlockSpec((r, c), lambda k: (0, 0))


def _bn_affine(s1, s2, g, b, n):
    mean = s1 / n
    var = s2 / n - mean * mean
    rs = lax.rsqrt(var + 1e-5)
    scale = rs * g[None, :]
    shift = b[None, :] - mean * scale
    return scale, shift


@functools.cache
def _make_edge_kernel(nchunks, rows_total):
    npsc = nchunks // 2
    rows_tile = rows_total // 16
    nblk = rows_tile // BLKG
    half = nblk // 2
    eb = BLKG * GRAN
    rpt = NPAD // 16
    zrows = rpt // 8
    mesh = plsc.VectorSubcoreMesh(core_axis_name="c", subcore_axis_name="s")

    @functools.partial(
        pl.kernel,
        out_type=jax.ShapeDtypeStruct((NPAD, nchunks * CHW), jnp.float32),
        mesh=mesh,
        scratch_types=[
            pltpu.VMEM_SHARED((NPAD, CHW), jnp.float32),
            pltpu.VMEM((BLKG, 2, GRAN), jnp.int32),
            pltpu.VMEM((BLKG, 2, GRAN), jnp.int32),
            pltpu.VMEM((eb, CHW), jnp.float32),
            pltpu.VMEM((eb, CHW), jnp.float32),
            pltpu.VMEM((zrows, CHW), jnp.float32),
            pltpu.SemaphoreType.DMA,
            pltpu.SemaphoreType.DMA,
            pltpu.SemaphoreType.DMA,
            pltpu.SemaphoreType.DMA,
            pltpu.SemaphoreType.DMA,
            pltpu.SemaphoreType.DMA,
        ],
        compiler_params=pltpu.CompilerParams(use_tc_tiling_on_sc=False),
    )
    def kern(table, edg, out, acc, idx_a, idx_b, gbuf_a, gbuf_b, zbuf,
             sem_ga, sem_gb, sem_sa, sem_sb, sem_ia, sem_ib):
        c = lax.axis_index("c")
        s = lax.axis_index("s")
        rbase = s * rpt
        ebase = s * rows_tile
        ncv = jnp.broadcast_to(jnp.int32(nchunks), (16,))
        zv = jnp.zeros((16,), jnp.float32)

        def zfill(i, carry):
            zbuf[i, pl.ds(0, CHW)] = zv
            return carry

        lax.fori_loop(0, zrows, zfill, 0)

        def fire_idx(bidx, idx, sem):
            row = ebase + bidx * BLKG
            pltpu.async_copy(edg.at[pl.ds(row, BLKG)], idx, sem)

        def finish_idx(bidx, idx, sem, chv):
            row = ebase + bidx * BLKG
            pltpu.make_async_copy(edg.at[pl.ds(row, BLKG)], idx, sem).wait()
            for j in range(BLKG):
                for l in range(GRAN // 16):
                    sl = pl.ds(l * 16, 16)
                    idx[j, 0, sl] = idx[j, 0, sl] + chv

        def fire_g(idx, gbuf, sem):
            for j in range(BLKG):
                pltpu.async_copy(table.at[idx.at[j, 0]],
                                 gbuf.at[pl.ds(j * GRAN, GRAN)], sem)

        def wait_g(idx, gbuf, sem):
            for j in range(BLKG):
                pltpu.make_async_copy(table.at[idx.at[j, 0]],
                                      gbuf.at[pl.ds(j * GRAN, GRAN)],
                                      sem).wait()

        def fire_s(idx, gbuf, sem):
            for j in range(BLKG):
                pltpu.async_copy(gbuf.at[pl.ds(j * GRAN, GRAN)],
                                 acc.at[idx.at[j, 1]], sem, add=True)

        def wait_s(idx, gbuf, sem):
            for j in range(BLKG):
                pltpu.make_async_copy(gbuf.at[pl.ds(j * GRAN, GRAN)],
                                      acc.at[idx.at[j, 1]], sem).wait()

        for ci in range(npsc):
            chunk = c * npsc + ci
            chv = jnp.broadcast_to(chunk, (16,)).astype(jnp.int32)
            for j in range(8):
                pltpu.sync_copy(zbuf, acc.at[pl.ds(rbase + j * zrows, zrows)])
            plsc.subcore_barrier()

            fire_idx(0, idx_a, sem_ia)
            finish_idx(0, idx_a, sem_ia, chv)
            fire_g(idx_a, gbuf_a, sem_ga)

            def pair(i, carry):
                wait_g(idx_a, gbuf_a, sem_ga)
                fire_s(idx_a, gbuf_a, sem_sa)

                @pl.when(i > 0)
                def _():
                    wait_s(idx_b, gbuf_b, sem_sb)

                fire_idx(2 * i + 1, idx_b, sem_ib)
                finish_idx(2 * i + 1, idx_b, sem_ib, chv)
                fire_g(idx_b, gbuf_b, sem_gb)
                wait_s(idx_a, gbuf_a, sem_sa)

                @pl.when(i < half - 1)
                def _():
                    fire_idx(2 * i + 2, idx_a, sem_ia)
                    finish_idx(2 * i + 2, idx_a, sem_ia, chv)
                    fire_g(idx_a, gbuf_a, sem_ga)

                wait_g(idx_b, gbuf_b, sem_gb)
                fire_s(idx_b, gbuf_b, sem_sb)
                return carry

            lax.fori_loop(0, half, pair, 0)
            wait_s(idx_b, gbuf_b, sem_sb)
            plsc.subcore_barrier()
            pltpu.sync_copy(
                acc.at[pl.ds(rbase, rpt)],
                out.at[pl.ds(rbase, rpt), pl.ds(chunk * CHW, CHW)])

    return kern


def kernel(x, params, edge_index, batch):
    p = params
    n = x.shape[0]
    ngroups = 128
    nb = _cdiv(n, ROWB)

    f32 = jnp.float32
    sd = jax.ShapeDtypeStruct

    src = edge_index[0]
    dst = edge_index[1]
    e = src.shape[0]
    epb = 16 * 2 * BLKG * GRAN
    epad = _cdiv(e, epb) * epb
    srcp = jnp.concatenate([src, jnp.zeros((epad - e,), jnp.int32)])
    dstp = jnp.concatenate([dst, jnp.full((epad - e,), n, jnp.int32)])
    rows_total = epad // GRAN
    dst2d = dstp.reshape(rows_total, GRAN)

    def _edg(nchunks):
        return jnp.stack([(srcp * nchunks).reshape(rows_total, GRAN), dst2d],
                         axis=1)

    hdim = p["W_in"].shape[1]
    h, s1, s2, hmn, hmx = pl.pallas_call(
        _k_input,
        grid=(nb,),
        in_specs=[_rows(6), _bcast(6, hdim), _bcast(1, hdim)],
        out_specs=[_rows(hdim), _bcast(1, hdim), _bcast(1, hdim),
                   _bcast(1, hdim), _bcast(1, hdim)],
        out_shape=[sd((n, hdim), f32), sd((1, hdim), f32), sd((1, hdim), f32),
                   sd((1, hdim), f32), sd((1, hdim), f32)],
    )(x, p["W_in"], p["b_in"][None])

    for i, lp in enumerate(p["layers"]):
        ch = h.shape[1]
        nchunks = 2 * ch // CHW
        ch2 = 2 * ch

        scale, shift = _bn_affine(s1, s2, lp["n_g"], lp["n_b"], n)
        vext = scale * jnp.where(scale > 0, hmx, hmn) + shift
        an = jnp.where(vext >= 0, vext, 0.01 * vext) if i == 0 \
            else jnp.tanh(vext)
        m = jnp.maximum(jnp.max(an), 0.0) + 1e-7

        hn, tab = pl.pallas_call(
            functools.partial(_k_preptab, i),
            grid=(nb,),
            in_specs=[_rows(ch), _bcast(1, ch), _bcast(1, ch), _bcast(1, ch)],
            out_specs=[_rows(ch), _rows(ch2)],
            out_shape=[sd((n, ch), f32), sd((n, ch2), f32)],
        )(h, scale, shift, jnp.broadcast_to(m, (1, ch)))

        ek = _make_edge_kernel(nchunks, rows_total)
        acc2 = ek(tab.reshape(n * nchunks, CHW), _edg(nchunks))

        z1, s1a, s2a = pl.pallas_call(
            functools.partial(_k_mlp1, ch),
            grid=(nb,),
            in_specs=[_rows(ch2), _rows(ch), _bcast(ch, ch2), _bcast(1, ch2)],
            out_specs=[_rows(ch2), _bcast(1, ch2), _bcast(1, ch2)],
            out_shape=[sd((n, ch2), f32), sd((1, ch2), f32),
                       sd((1, ch2), f32)],
        )(acc2, hn, lp["m1W"], lp["m1b"][None])

        sc2, sh2 = _bn_affine(s1a, s2a, lp["m1g"], lp["m1be"], n)
        z2, s1b, s2b = pl.pallas_call(
            _k_mlp2,
            grid=(nb,),
            in_specs=[_rows(ch2), _bcast(1, ch2), _bcast(1, ch2),
                      _bcast(ch2, ch2), _bcast(1, ch2)],
            out_specs=[_rows(ch2), _bcast(1, ch2), _bcast(1, ch2)],
            out_shape=[sd((n, ch2), f32), sd((1, ch2), f32),
                       sd((1, ch2), f32)],
        )(z1, sc2, sh2, lp["m2W"], lp["m2b"][None])

        sc3, sh3 = _bn_affine(s1b, s2b, lp["m2g"], lp["m2be"], n)
        cho = ch // 2
        h, s1, s2, hmn, hmx = pl.pallas_call(
            _k_mlp3enc,
            grid=(nb,),
            in_specs=[_rows(ch2), _bcast(1, ch2), _bcast(1, ch2),
                      _bcast(ch2, ch), _bcast(1, ch), _rows(ch),
                      _bcast(ch, cho), _bcast(1, cho)],
            out_specs=[_rows(cho), _bcast(1, cho), _bcast(1, cho),
                       _bcast(1, cho), _bcast(1, cho)],
            out_shape=[sd((n, cho), f32), sd((1, cho), f32),
                       sd((1, cho), f32), sd((1, cho), f32),
                       sd((1, cho), f32)],
        )(z2, sc3, sh3, lp["m3W"], lp["m3b"][None], h, lp["eW"],
          lp["eb"][None])

    cho = h.shape[1]
    ps, cnt = pl.pallas_call(
        functools.partial(_k_pool, ngroups),
        grid=(nb,),
        in_specs=[pl.BlockSpec((1, 1, ROWB), lambda k: (k, 0, 0)), _rows(cho)],
        out_specs=[_bcast(ngroups, cho), _bcast(ngroups, 1)],
        out_shape=[sd((ngroups, cho), f32), sd((ngroups, 1), f32)],
    )(batch.astype(jnp.int32).reshape(nb, 1, ROWB), h)

    out = pl.pallas_call(
        _k_final,
        in_specs=[_bcast(ngroups, cho), _bcast(ngroups, 1),
                  _bcast(cho, 2), _bcast(1, 2)],
        out_specs=pl.BlockSpec((ngroups, 2), lambda k: (0, 0)),
        out_shape=sd((ngroups, 2), f32),
        grid=(1,),
    )(ps, cnt, p["W_lin"], p["b_lin"][None])
    return out


# speedup vs baseline: 1.0195x; 1.0047x over previous
import functools

import jax
import jax.numpy as jnp
from jax import lax
from jax.experimental import pallas as pl
from jax.experimental.pallas import tpu as pltpu
from jax.experimental.pallas import tpu_sc as plsc

ROWB = 1000
CHW = 16
GRAN = 128
GG = 256
BLKG = 14
BLKGG = 7
NPAD = 50048


def _cdiv(a, b):
    return (a + b - 1) // b


def _k_input(x_ref, w_ref, b_ref, h_ref, s1_ref, s2_ref, mn_ref, mx_ref):
    h = jnp.dot(x_ref[...], w_ref[...], preferred_element_type=jnp.float32)
    h = h + b_ref[...]
    h_ref[...] = h

    @pl.when(pl.program_id(0) == 0)
    def _():
        s1_ref[...] = jnp.zeros_like(s1_ref)
        s2_ref[...] = jnp.zeros_like(s2_ref)
        mn_ref[...] = jnp.full_like(mn_ref, jnp.inf)
        mx_ref[...] = jnp.full_like(mx_ref, -jnp.inf)

    s1_ref[...] += jnp.sum(h, axis=0, keepdims=True)
    s2_ref[...] += jnp.sum(h * h, axis=0, keepdims=True)
    mn_ref[...] = jnp.minimum(mn_ref[...], jnp.min(h, axis=0, keepdims=True))
    mx_ref[...] = jnp.maximum(mx_ref[...], jnp.max(h, axis=0, keepdims=True))


def _k_preptab(act_i, h_ref, sc_ref, sh_ref, m_ref, hn_ref, tab_ref):
    v = h_ref[...] * sc_ref[...] + sh_ref[...]
    if act_i == 0:
        hn = jnp.where(v >= 0, v, 0.01 * v)
    else:
        hn = jnp.tanh(v)
    hn_ref[...] = hn
    r = jnp.maximum(hn, 0.0) + 1e-7
    t = jnp.exp(r - m_ref[...])
    u = r * t
    tab_ref[...] = jnp.concatenate([t, u], axis=1)


def _k_mlp1(ch, acc_ref, hn_ref, w_ref, b_ref, z_ref, s1_ref, s2_ref):
    a = acc_ref[...]
    den = a[:, 0:ch]
    num = a[:, ch:2 * ch]
    safe = jnp.where(den > 0, den, 1.0)
    agg = jnp.where(den > 0, num / safe, 0.0)
    out = agg + hn_ref[...]
    z = jnp.dot(out, w_ref[...], preferred_element_type=jnp.float32)
    z = z + b_ref[...]
    z_ref[...] = z

    @pl.when(pl.program_id(0) == 0)
    def _():
        s1_ref[...] = jnp.zeros_like(s1_ref)
        s2_ref[...] = jnp.zeros_like(s2_ref)

    s1_ref[...] += jnp.sum(z, axis=0, keepdims=True)
    s2_ref[...] += jnp.sum(z * z, axis=0, keepdims=True)


def _k_mlp2(z1_ref, sc_ref, sh_ref, w_ref, b_ref, z_ref, s1_ref, s2_ref):
    a = jnp.maximum(z1_ref[...] * sc_ref[...] + sh_ref[...], 0.0)
    z = jnp.dot(a, w_ref[...], preferred_element_type=jnp.float32)
    z = z + b_ref[...]
    z_ref[...] = z

    @pl.when(pl.program_id(0) == 0)
    def _():
        s1_ref[...] = jnp.zeros_like(s1_ref)
        s2_ref[...] = jnp.zeros_like(s2_ref)

    s1_ref[...] += jnp.sum(z, axis=0, keepdims=True)
    s2_ref[...] += jnp.sum(z * z, axis=0, keepdims=True)


def _k_mlp3enc(z2_ref, sc_ref, sh_ref, w3_ref, b3_ref, h_ref, we_ref, be_ref,
               ho_ref, s1_ref, s2_ref, mn_ref, mx_ref):
    a = jnp.maximum(z2_ref[...] * sc_ref[...] + sh_ref[...], 0.0)
    z3 = jnp.dot(a, w3_ref[...], preferred_element_type=jnp.float32)
    hm = h_ref[...] + z3 + b3_ref[...]
    ho = jnp.dot(hm, we_ref[...], preferred_element_type=jnp.float32)
    ho = ho + be_ref[...]
    ho_ref[...] = ho

    @pl.when(pl.program_id(0) == 0)
    def _():
        s1_ref[...] = jnp.zeros_like(s1_ref)
        s2_ref[...] = jnp.zeros_like(s2_ref)
        mn_ref[...] = jnp.full_like(mn_ref, jnp.inf)
        mx_ref[...] = jnp.full_like(mx_ref, -jnp.inf)

    s1_ref[...] += jnp.sum(ho, axis=0, keepdims=True)
    s2_ref[...] += jnp.sum(ho * ho, axis=0, keepdims=True)
    mn_ref[...] = jnp.minimum(mn_ref[...], jnp.min(ho, axis=0, keepdims=True))
    mx_ref[...] = jnp.maximum(mx_ref[...], jnp.max(ho, axis=0, keepdims=True))


def _k_pool(ngroups, bat_ref, h_ref, ps_ref, cnt_ref):
    @pl.when(pl.program_id(0) == 0)
    def _():
        ps_ref[...] = jnp.zeros_like(ps_ref)
        cnt_ref[...] = jnp.zeros_like(cnt_ref)

    gi = lax.broadcasted_iota(jnp.int32, (ngroups, ROWB), 0)
    oh = (gi == bat_ref[0]).astype(jnp.float32)
    ps_ref[...] += jnp.dot(oh, h_ref[...], preferred_element_type=jnp.float32)
    cnt_ref[...] += jnp.sum(oh, axis=1, keepdims=True)


def _k_final(ps_ref, cnt_ref, w_ref, b_ref, o_ref):
    pooled = ps_ref[...] / jnp.maximum(cnt_ref[...], 1.0)
    o = jnp.dot(pooled, w_ref[...], preferred_element_type=jnp.float32)
    o_ref[...] = o + b_ref[...]


def _rows(n):
    return pl.BlockSpec((ROWB, n), lambda k: (k, 0))


def _bcast(r, c):
    return pl.BlockSpec((r, c), lambda k: (0, 0))


def _bn_affine(s1, s2, g, b, n):
    mean = s1 / n
    var = s2 / n - mean * mean
    rs = lax.rsqrt(var + 1e-5)
    scale = rs * g[None, :]
    shift = b[None, :] - mean * scale
    return scale, shift


@functools.cache
def _make_edge_kernel(nchunks, rows_total):
    npsc = nchunks // 2
    rows_tile = rows_total // 16
    nblk = rows_tile // BLKG
    half = nblk // 2
    eb = BLKG * GRAN
    rpt = NPAD // 16
    zrows = rpt // 8
    mesh = plsc.VectorSubcoreMesh(core_axis_name="c", subcore_axis_name="s")

    @functools.partial(
        pl.kernel,
        out_type=jax.ShapeDtypeStruct((NPAD, nchunks * CHW), jnp.float32),
        mesh=mesh,
        scratch_types=[
            pltpu.VMEM_SHARED((NPAD, CHW), jnp.float32),
            pltpu.VMEM((BLKGG, GG), jnp.int32),
            pltpu.VMEM((BLKG, GRAN), jnp.int32),
            pltpu.VMEM((BLKGG, GG), jnp.int32),
            pltpu.VMEM((BLKG, GRAN), jnp.int32),
            pltpu.VMEM((eb, CHW), jnp.float32),
            pltpu.VMEM((eb, CHW), jnp.float32),
            pltpu.VMEM((zrows, CHW), jnp.float32),
            pltpu.SemaphoreType.DMA,
            pltpu.SemaphoreType.DMA,
            pltpu.SemaphoreType.DMA,
            pltpu.SemaphoreType.DMA,
            pltpu.SemaphoreType.DMA,
            pltpu.SemaphoreType.DMA,
        ],
        compiler_params=pltpu.CompilerParams(use_tc_tiling_on_sc=False),
    )
    def kern(table, srcm, dst2d, out, acc, sidx_a, didx_a, sidx_b, didx_b,
             gbuf_a, gbuf_b, zbuf, sem_ga, sem_gb, sem_sa, sem_sb,
             sem_ia, sem_ib):
        c = lax.axis_index("c")
        s = lax.axis_index("s")
        rbase = s * rpt
        ebase = s * rows_tile
        gbase = s * (rows_tile // 2)
        zv = jnp.zeros((16,), jnp.float32)

        def zfill(i, carry):
            zbuf[i, pl.ds(0, CHW)] = zv
            return carry

        lax.fori_loop(0, zrows, zfill, 0)

        def fire_idx(bidx, sidx, didx, sem):
            grow = gbase + bidx * BLKGG
            row = ebase + bidx * BLKG
            pltpu.async_copy(srcm.at[pl.ds(grow, BLKGG)], sidx, sem)
            pltpu.async_copy(dst2d.at[pl.ds(row, BLKG)], didx, sem)

        def finish_idx(bidx, sidx, didx, sem, chv):
            grow = gbase + bidx * BLKGG
            row = ebase + bidx * BLKG
            pltpu.make_async_copy(srcm.at[pl.ds(grow, BLKGG)], sidx,
                                  sem).wait()
            pltpu.make_async_copy(dst2d.at[pl.ds(row, BLKG)], didx,
                                  sem).wait()
            for j in range(BLKGG):
                for l in range(GG // 16):
                    sl = pl.ds(l * 16, 16)
                    sidx[j, sl] = sidx[j, sl] + chv

        def fire_g(sidx, gbuf, sem):
            for j in range(BLKGG):
                pltpu.async_copy(table.at[sidx.at[j]],
                                 gbuf.at[pl.ds(j * GG, GG)], sem)

        def wait_g(sidx, gbuf, sem):
            for j in range(BLKGG):
                pltpu.make_async_copy(table.at[sidx.at[j]],
                                      gbuf.at[pl.ds(j * GG, GG)],
                                      sem).wait()

        def fire_s(didx, gbuf, sem):
            for j in range(BLKG):
                pltpu.async_copy(gbuf.at[pl.ds(j * GRAN, GRAN)],
                                 acc.at[didx.at[j]], sem, add=True)

        def wait_s(didx, gbuf, sem):
            for j in range(BLKG):
                pltpu.make_async_copy(gbuf.at[pl.ds(j * GRAN, GRAN)],
                                      acc.at[didx.at[j]], sem).wait()

        for ci in range(npsc):
            chunk = c * npsc + ci
            chv = jnp.broadcast_to(chunk, (16,)).astype(jnp.int32)
            for j in range(8):
                pltpu.sync_copy(zbuf, acc.at[pl.ds(rbase + j * zrows, zrows)])
            plsc.subcore_barrier()

            fire_idx(0, sidx_a, didx_a, sem_ia)
            finish_idx(0, sidx_a, didx_a, sem_ia, chv)
            fire_g(sidx_a, gbuf_a, sem_ga)

            def pair(i, carry):
                wait_g(sidx_a, gbuf_a, sem_ga)
                fire_s(didx_a, gbuf_a, sem_sa)

                @pl.when(i > 0)
                def _():
                    wait_s(didx_b, gbuf_b, sem_sb)

                fire_idx(2 * i + 1, sidx_b, didx_b, sem_ib)
                finish_idx(2 * i + 1, sidx_b, didx_b, sem_ib, chv)
                fire_g(sidx_b, gbuf_b, sem_gb)
                wait_s(didx_a, gbuf_a, sem_sa)

                @pl.when(i < half - 1)
                def _():
                    fire_idx(2 * i + 2, sidx_a, didx_a, sem_ia)
                    finish_idx(2 * i + 2, sidx_a, didx_a, sem_ia, chv)
                    fire_g(sidx_a, gbuf_a, sem_ga)

                wait_g(sidx_b, gbuf_b, sem_gb)
                fire_s(didx_b, gbuf_b, sem_sb)
                return carry

            lax.fori_loop(0, half, pair, 0)
            wait_s(didx_b, gbuf_b, sem_sb)
            plsc.subcore_barrier()
            pltpu.sync_copy(
                acc.at[pl.ds(rbase, rpt)],
                out.at[pl.ds(rbase, rpt), pl.ds(chunk * CHW, CHW)])

    return kern


def kernel(x, params, edge_index, batch):
    p = params
    n = x.shape[0]
    ngroups = 128
    nb = _cdiv(n, ROWB)

    f32 = jnp.float32
    sd = jax.ShapeDtypeStruct

    src = edge_index[0]
    dst = edge_index[1]
    e = src.shape[0]
    epb = 16 * 2 * BLKG * GRAN
    epad = _cdiv(e, epb) * epb
    srcp = jnp.concatenate([src, jnp.zeros((epad - e,), jnp.int32)])
    dstp = jnp.concatenate([dst, jnp.full((epad - e,), n, jnp.int32)])
    rows_total = epad // GRAN
    dst2d = dstp.reshape(rows_total, GRAN)

    def _srcm(nchunks):
        return (srcp * nchunks).reshape(epad // GG, GG)

    hdim = p["W_in"].shape[1]
    h, s1, s2, hmn, hmx = pl.pallas_call(
        _k_input,
        grid=(nb,),
        in_specs=[_rows(6), _bcast(6, hdim), _bcast(1, hdim)],
        out_specs=[_rows(hdim), _bcast(1, hdim), _bcast(1, hdim),
                   _bcast(1, hdim), _bcast(1, hdim)],
        out_shape=[sd((n, hdim), f32), sd((1, hdim), f32), sd((1, hdim), f32),
                   sd((1, hdim), f32), sd((1, hdim), f32)],
    )(x, p["W_in"], p["b_in"][None])

    for i, lp in enumerate(p["layers"]):
        ch = h.shape[1]
        nchunks = 2 * ch // CHW
        ch2 = 2 * ch

        scale, shift = _bn_affine(s1, s2, lp["n_g"], lp["n_b"], n)
        vext = scale * jnp.where(scale > 0, hmx, hmn) + shift
        an = jnp.where(vext >= 0, vext, 0.01 * vext) if i == 0 \
            else jnp.tanh(vext)
        m = jnp.maximum(jnp.max(an), 0.0) + 1e-7

        hn, tab = pl.pallas_call(
            functools.partial(_k_preptab, i),
            grid=(nb,),
            in_specs=[_rows(ch), _bcast(1, ch), _bcast(1, ch), _bcast(1, ch)],
            out_specs=[_rows(ch), _rows(ch2)],
            out_shape=[sd((n, ch), f32), sd((n, ch2), f32)],
        )(h, scale, shift, jnp.broadcast_to(m, (1, ch)))

        ek = _make_edge_kernel(nchunks, rows_total)
        acc2 = ek(tab.reshape(n * nchunks, CHW), _srcm(nchunks), dst2d)

        z1, s1a, s2a = pl.pallas_call(
            functools.partial(_k_mlp1, ch),
            grid=(nb,),
            in_specs=[_rows(ch2), _rows(ch), _bcast(ch, ch2), _bcast(1, ch2)],
            out_specs=[_rows(ch2), _bcast(1, ch2), _bcast(1, ch2)],
            out_shape=[sd((n, ch2), f32), sd((1, ch2), f32),
                       sd((1, ch2), f32)],
        )(acc2, hn, lp["m1W"], lp["m1b"][None])

        sc2, sh2 = _bn_affine(s1a, s2a, lp["m1g"], lp["m1be"], n)
        z2, s1b, s2b = pl.pallas_call(
            _k_mlp2,
            grid=(nb,),
            in_specs=[_rows(ch2), _bcast(1, ch2), _bcast(1, ch2),
                      _bcast(ch2, ch2), _bcast(1, ch2)],
            out_specs=[_rows(ch2), _bcast(1, ch2), _bcast(1, ch2)],
            out_shape=[sd((n, ch2), f32), sd((1, ch2), f32),
                       sd((1, ch2), f32)],
        )(z1, sc2, sh2, lp["m2W"], lp["m2b"][None])

        sc3, sh3 = _bn_affine(s1b, s2b, lp["m2g"], lp["m2be"], n)
        cho = ch // 2
        h, s1, s2, hmn, hmx = pl.pallas_call(
            _k_mlp3enc,
            grid=(nb,),
            in_specs=[_rows(ch2), _bcast(1, ch2), _bcast(1, ch2),
                      _bcast(ch2, ch), _bcast(1, ch), _rows(ch),
                      _bcast(ch, cho), _bcast(1, cho)],
            out_specs=[_rows(cho), _bcast(1, cho), _bcast(1, cho),
                       _bcast(1, cho), _bcast(1, cho)],
            out_shape=[sd((n, cho), f32), sd((1, cho), f32),
                       sd((1, cho), f32), sd((1, cho), f32),
                       sd((1, cho), f32)],
        )(z2, sc3, sh3, lp["m3W"], lp["m3b"][None], h, lp["eW"],
          lp["eb"][None])

    cho = h.shape[1]
    ps, cnt = pl.pallas_call(
        functools.partial(_k_pool, ngroups),
        grid=(nb,),
        in_specs=[pl.BlockSpec((1, 1, ROWB), lambda k: (k, 0, 0)), _rows(cho)],
        out_specs=[_bcast(ngroups, cho), _bcast(ngroups, 1)],
        out_shape=[sd((ngroups, cho), f32), sd((ngroups, 1), f32)],
    )(batch.astype(jnp.int32).reshape(nb, 1, ROWB), h)

    out = pl.pallas_call(
        _k_final,
        in_specs=[_bcast(ngroups, cho), _bcast(ngroups, 1),
                  _bcast(cho, 2), _bcast(1, 2)],
        out_specs=pl.BlockSpec((ngroups, 2), lambda k: (0, 0)),
        out_shape=sd((ngroups, 2), f32),
        grid=(1,),
    )(ps, cnt, p["W_lin"], p["b_lin"][None])
    return out
